# baseline (speedup 1.0000x reference)
import functools

import jax
import jax.numpy as jnp
from jax import lax
from jax.experimental import pallas as pl
from jax.experimental.pallas import tpu as pltpu
from jax.experimental.pallas import tpu_sc as plsc

L = 16


def kernel(user_ids, item_ids, user_emb, item_emb, user_bias, item_bias, W, b):
    B = user_ids.shape[0]
    H = user_emb.shape[1]
    info = plsc.get_sparse_core_info()
    NC, NS = info.num_cores, info.num_subcores
    NW = NC * NS
    bpw = B // NW
    CH = 128
    nch = bpw // CH
    ngrp = bpw // L

    uids = user_ids.astype(jnp.int32).reshape(NW, nch, CH)
    iids = item_ids.astype(jnp.int32).reshape(NW, nch, CH)
    ub_tbl = user_bias.reshape(-1)
    ib_tbl = item_bias.reshape(-1)
    wvec = jnp.concatenate(
        [W.reshape(-1), b.reshape(-1), jnp.zeros((6,), jnp.float32)]
    )

    mesh = plsc.VectorSubcoreMesh(core_axis_name="c", subcore_axis_name="s")

    @functools.partial(
        pl.kernel,
        out_type=jax.ShapeDtypeStruct((B,), jnp.float32),
        mesh=mesh,
        scratch_types=[
            pltpu.VMEM((nch, CH), jnp.int32),
            pltpu.VMEM((nch, CH), jnp.int32),
            pltpu.VMEM((bpw, H), jnp.float32),
            pltpu.VMEM((bpw, H), jnp.float32),
            pltpu.VMEM((bpw,), jnp.float32),
            pltpu.VMEM((bpw,), jnp.float32),
            pltpu.VMEM((bpw,), jnp.float32),
            pltpu.VMEM((H + 8,), jnp.float32),
            pltpu.SemaphoreType.DMA,
        ],
        compiler_params=pltpu.CompilerParams(
            needs_layout_passes=False, use_tc_tiling_on_sc=False
        ),
    )
    def biased_gmf_sc(uids_hbm, iids_hbm, ue_hbm, ie_hbm, ub_hbm, ib_hbm,
                      w_hbm, out_hbm,
                      uidx, iidx, uev, iev, ubv, ibv, outv, wv, sem):
        wid = lax.axis_index("s") * NC + lax.axis_index("c")
        base = wid * bpw

        pltpu.sync_copy(uids_hbm.at[wid], uidx)
        pltpu.sync_copy(iids_hbm.at[wid], iidx)
        pltpu.sync_copy(w_hbm, wv)

        cps = []
        for c in range(nch):
            rows = pl.ds(c * CH, CH)
            cps.append(pltpu.async_copy(ue_hbm.at[uidx.at[c]], uev.at[rows], sem))
            cps.append(pltpu.async_copy(ie_hbm.at[iidx.at[c]], iev.at[rows], sem))
            cps.append(pltpu.async_copy(ub_hbm.at[uidx.at[c]], ubv.at[rows], sem))
            cps.append(pltpu.async_copy(ib_hbm.at[iidx.at[c]], ibv.at[rows], sem))
        for cp in cps:
            cp.wait()

        wc = [wv[pl.ds(c * L, L)] for c in range(H // L)]
        wtail = wv[pl.ds(H - 8, L)]
        w_bias = wtail[8]
        b_lin = wtail[9]
        lane = lax.iota(jnp.int32, L)

        def group_body(g, carry):
            r0 = g * L
            res = jnp.zeros((L,), jnp.float32)
            for j in range(L):
                r = r0 + j
                s = jnp.zeros((L,), jnp.float32)
                for c in range(H // L):
                    hs = pl.ds(c * L, L)
                    s = s + uev[r, hs] * iev[r, hs] * wc[c]
                res = jnp.where(lane == j, jnp.sum(s), res)
            ubg = ubv[pl.ds(r0, L)]
            ibg = ibv[pl.ds(r0, L)]
            outv[pl.ds(r0, L)] = res + (ubg + ibg) * w_bias + b_lin
            return carry

        lax.fori_loop(0, ngrp, group_body, 0)

        pltpu.sync_copy(outv, out_hbm.at[pl.ds(base, bpw)])

    return biased_gmf_sc(uids, iids, user_emb, item_emb, ub_tbl, ib_tbl, wvec)
